# Initial kernel scaffold; baseline (speedup 1.0000x reference)
import jax
import jax.numpy as jnp
from jax.experimental import pallas as pl


def kernel(h, x, edges, emb_in_W, emb_in_b, edge_W1, edge_b1, edge_W2, edge_b2, node_W1, node_b1, node_W2, node_b2, coord_W1, coord_b1, coord_W2, emb_out_W, emb_out_b, head_W1, head_b1, head_W2, head_b2):
    raise NotImplementedError("write your pallas kernel here")


# speedup vs baseline: 3.8224x; 3.8224x over previous
import functools

import jax
import jax.numpy as jnp
from jax import lax
from jax.experimental import pallas as pl
from jax.experimental.pallas import tpu as pltpu
from jax.experimental.pallas import tpu_sc as plsc

N = 10000
E = 320000
H = 64
L = 4
D_IN = 128
D_OUT = 20

NC = 2
NS = 16
NW = NC * NS
EPW = E // NW
CH = 400
NCHUNK = EPW // CH
SUB = 80
NSUB = CH // SUB
SSC = 40
NSSC = CH // SSC

E4 = E // 4
BE4 = 1600
NPAD = 10240
NB = 2048

_f32 = jnp.float32


def _silu(v):
    return v * jax.nn.sigmoid(v)


def _bdot(x, w):
    return jnp.dot(x.astype(jnp.bfloat16), w.astype(jnp.bfloat16),
                   preferred_element_type=_f32)


def _init_body(h_ref, w_ref, b_ref, wa_ref, wb_ref, hh_ref, a_ref, bb_ref):
    hh = _bdot(h_ref[...], w_ref[...]) + b_ref[...]
    hh_ref[...] = hh
    a_ref[...] = _bdot(hh, wa_ref[...])
    bb_ref[...] = _bdot(hh, wb_ref[...])


def _init_call(h, w, b, wa, wb):
    return pl.pallas_call(
        _init_body,
        grid=(NPAD // NB,),
        in_specs=[
            pl.BlockSpec((NB, D_IN), lambda i: (i, 0)),
            pl.BlockSpec((D_IN, H), lambda i: (0, 0)),
            pl.BlockSpec((1, H), lambda i: (0, 0)),
            pl.BlockSpec((H, H), lambda i: (0, 0)),
            pl.BlockSpec((H, H), lambda i: (0, 0)),
        ],
        out_specs=[
            pl.BlockSpec((NB, H), lambda i: (i, 0)),
            pl.BlockSpec((NB, H), lambda i: (i, 0)),
            pl.BlockSpec((NB, H), lambda i: (i, 0)),
        ],
        out_shape=[jax.ShapeDtypeStruct((NPAD, H), _f32)] * 3,
    )(h, w, b, wa, wb)


def _edge_body(ar_ref, bc_ref, rad_ref, r4_ref, b1_ref, w2_ref, b2_ref,
               cw1_ref, cb1_ref, cw2_ref, m_ref, c_ref):
    z = (ar_ref[...] + bc_ref[...]
         + _bdot(rad_ref[...], r4_ref[...])
         + b1_ref[...])
    m1 = _silu(z)
    m2 = _silu(_bdot(m1, w2_ref[...]) + b2_ref[...])
    c1 = _silu(_bdot(m2, cw1_ref[...]) + cb1_ref[...])
    m_ref[...] = m2
    c_ref[...] = _bdot(c1, cw2_ref[...])


def _edge_call(ar4, bc4, rad4, r4, b1t, w2bd, b2t, cw1bd, cb1t, cw2bd):
    return pl.pallas_call(
        _edge_body,
        grid=(E4 // BE4,),
        in_specs=[
            pl.BlockSpec((BE4, 256), lambda i: (i, 0)),
            pl.BlockSpec((BE4, 256), lambda i: (i, 0)),
            pl.BlockSpec((BE4, 4), lambda i: (i, 0)),
            pl.BlockSpec((4, 256), lambda i: (0, 0)),
            pl.BlockSpec((1, 256), lambda i: (0, 0)),
            pl.BlockSpec((256, 256), lambda i: (0, 0)),
            pl.BlockSpec((1, 256), lambda i: (0, 0)),
            pl.BlockSpec((256, 256), lambda i: (0, 0)),
            pl.BlockSpec((1, 256), lambda i: (0, 0)),
            pl.BlockSpec((256, 4), lambda i: (0, 0)),
        ],
        out_specs=[
            pl.BlockSpec((BE4, 256), lambda i: (i, 0)),
            pl.BlockSpec((BE4, 4), lambda i: (i, 0)),
        ],
        out_shape=[
            jax.ShapeDtypeStruct((E4, 256), _f32),
            jax.ShapeDtypeStruct((E4, 4), _f32),
        ],
    )(ar4, bc4, rad4, r4, b1t, w2bd, b2t, cw1bd, cb1t, cw2bd)


def _node_body(hh_ref, coord_ref, agg_ref, seg_ref, nw1_ref, nb1_ref, nw2_ref,
               nb2_ref, wa_ref, wb_ref, hho_ref, coordo_ref, a_ref, bb_ref):
    agg = agg_ref[0] + agg_ref[1]
    hh = hh_ref[...]
    zin = jnp.concatenate([hh, agg], axis=1)
    u = _silu(_bdot(zin, nw1_ref[...]) + nb1_ref[...])
    hhn = hh + _bdot(u, nw2_ref[...]) + nb2_ref[...]
    hho_ref[...] = hhn
    seg = seg_ref[0] + seg_ref[1]
    cnt = jnp.clip(seg[:, 3:4], 1.0, None)
    coordo_ref[...] = coord_ref[...] + seg[:, 0:3] / cnt
    a_ref[...] = _bdot(hhn, wa_ref[...])
    bb_ref[...] = _bdot(hhn, wb_ref[...])


def _node_call(hh, coord3, aggp, segp, nw1, nb1, nw2, nb2, wa, wb):
    return pl.pallas_call(
        _node_body,
        grid=(NPAD // NB,),
        in_specs=[
            pl.BlockSpec((NB, H), lambda i: (i, 0)),
            pl.BlockSpec((NB, 3), lambda i: (i, 0)),
            pl.BlockSpec((2, NB, H), lambda i: (0, i, 0)),
            pl.BlockSpec((2, NB, 16), lambda i: (0, i, 0)),
            pl.BlockSpec((2 * H, H), lambda i: (0, 0)),
            pl.BlockSpec((1, H), lambda i: (0, 0)),
            pl.BlockSpec((H, H), lambda i: (0, 0)),
            pl.BlockSpec((1, H), lambda i: (0, 0)),
            pl.BlockSpec((H, H), lambda i: (0, 0)),
            pl.BlockSpec((H, H), lambda i: (0, 0)),
        ],
        out_specs=[
            pl.BlockSpec((NB, H), lambda i: (i, 0)),
            pl.BlockSpec((NB, 3), lambda i: (i, 0)),
            pl.BlockSpec((NB, H), lambda i: (i, 0)),
            pl.BlockSpec((NB, H), lambda i: (i, 0)),
        ],
        out_shape=[
            jax.ShapeDtypeStruct((NPAD, H), _f32),
            jax.ShapeDtypeStruct((NPAD, 3), _f32),
            jax.ShapeDtypeStruct((NPAD, H), _f32),
            jax.ShapeDtypeStruct((NPAD, H), _f32),
        ],
    )(hh, coord3, aggp, segp, nw1, nb1, nw2, nb2, wa, wb)


def _last_body(hh_ref, agg_ref, nw1_ref, nb1_ref, nw2_ref, nb2_ref, eow_ref,
               eob_ref, hw1_ref, hb1_ref, hw2_ref, hb2_ref, out_ref):
    agg = agg_ref[0] + agg_ref[1]
    hh = hh_ref[...]
    zin = jnp.concatenate([hh, agg], axis=1)
    u = _silu(_bdot(zin, nw1_ref[...]) + nb1_ref[...])
    hhn = hh + _bdot(u, nw2_ref[...]) + nb2_ref[...]
    g = _bdot(hhn, eow_ref[...]) + eob_ref[...]
    r = jnp.maximum(_bdot(g, hw1_ref[...]) + hb1_ref[...], 0.0)
    out_ref[...] = _bdot(r, hw2_ref[...]) + hb2_ref[...]


def _last_call(hh, aggp, nw1, nb1, nw2, nb2, eow, eob, hw1, hb1, hw2, hb2):
    return pl.pallas_call(
        _last_body,
        grid=(NPAD // NB,),
        in_specs=[
            pl.BlockSpec((NB, H), lambda i: (i, 0)),
            pl.BlockSpec((2, NB, H), lambda i: (0, i, 0)),
            pl.BlockSpec((2 * H, H), lambda i: (0, 0)),
            pl.BlockSpec((1, H), lambda i: (0, 0)),
            pl.BlockSpec((H, H), lambda i: (0, 0)),
            pl.BlockSpec((1, H), lambda i: (0, 0)),
            pl.BlockSpec((H, H), lambda i: (0, 0)),
            pl.BlockSpec((1, H), lambda i: (0, 0)),
            pl.BlockSpec((H, H), lambda i: (0, 0)),
            pl.BlockSpec((1, H), lambda i: (0, 0)),
            pl.BlockSpec((H, D_OUT), lambda i: (0, 0)),
            pl.BlockSpec((1, D_OUT), lambda i: (0, 0)),
        ],
        out_specs=[pl.BlockSpec((NB, D_OUT), lambda i: (i, 0))],
        out_shape=[jax.ShapeDtypeStruct((NPAD, D_OUT), _f32)],
    )(hh, aggp, nw1, nb1, nw2, nb2, eow, eob, hw1, hb1, hw2, hb2)[0]


@functools.cache
def _mesh():
    return plsc.VectorSubcoreMesh(core_axis_name="c", subcore_axis_name="s",
                                  num_cores=NC, num_subcores=NS)


_SC_PARAMS = pltpu.CompilerParams(needs_layout_passes=False,
                                  use_tc_tiling_on_sc=False)


def _sc_gather_body(a_hbm, b_hbm, coordf_hbm, row_hbm, col_hbm,
                    ar_out, bc_out, rad_out,
                    idxr_v, idxc_v, bufa_v, bufb_v, coord_v, rad_v, sem):
    wid = lax.axis_index("s") * NC + lax.axis_index("c")
    base = wid * EPW
    pltpu.sync_copy(coordf_hbm, coord_v)

    def chunk(ci, carry):
        off = base + ci * CH
        for j in range(NSUB):
            pltpu.sync_copy(row_hbm.at[pl.ds(off + j * SUB, SUB)], idxr_v.at[j])
            pltpu.sync_copy(col_hbm.at[pl.ds(off + j * SUB, SUB)], idxc_v.at[j])
        cps = []
        for j in range(NSUB):
            cps.append(pltpu.async_copy(
                a_hbm.at[idxr_v.at[j]], bufa_v.at[pl.ds(j * SUB, SUB)], sem))
            cps.append(pltpu.async_copy(
                b_hbm.at[idxc_v.at[j]], bufb_v.at[pl.ds(j * SUB, SUB)], sem))
        for g in range(CH // 16):
            j = (g * 16) // SUB
            k = (g * 16) % SUB
            r16 = idxr_v[j, pl.ds(k, 16)]
            c16 = idxc_v[j, pl.ds(k, 16)]
            r3 = r16 * 3
            c3 = c16 * 3
            acc = jnp.zeros((16,), _f32)
            for d in range(3):
                dsp = jnp.full((16,), d, jnp.int32)
                pr = plsc.load_gather(coord_v, [r3 + dsp])
                pc = plsc.load_gather(coord_v, [c3 + dsp])
                df = pr - pc
                acc = acc + df * df
            rad_v[pl.ds(g * 16, 16)] = acc
        for cp in cps:
            cp.wait()
        pltpu.sync_copy(bufa_v, ar_out.at[pl.ds(off, CH)])
        pltpu.sync_copy(bufb_v, bc_out.at[pl.ds(off, CH)])
        pltpu.sync_copy(rad_v, rad_out.at[pl.ds(off, CH)])
        return carry

    lax.fori_loop(0, NCHUNK, chunk, 0)


def _sc_gather_call(a, b, coordf, row1, col1):
    f = pl.kernel(
        _sc_gather_body,
        out_type=[
            jax.ShapeDtypeStruct((E, H), _f32),
            jax.ShapeDtypeStruct((E, H), _f32),
            jax.ShapeDtypeStruct((E,), _f32),
        ],
        mesh=_mesh(),
        compiler_params=_SC_PARAMS,
        scratch_types=[
            pltpu.VMEM((NSUB, SUB), jnp.int32),
            pltpu.VMEM((NSUB, SUB), jnp.int32),
            pltpu.VMEM((CH, H), _f32),
            pltpu.VMEM((CH, H), _f32),
            pltpu.VMEM((NPAD * 3,), _f32),
            pltpu.VMEM((CH,), _f32),
            pltpu.SemaphoreType.DMA,
        ],
    )
    return f(a, b, coordf, row1, col1)


def _sc_scatter_body(m_hbm, c_hbm, row_hbm, col_hbm, coordf_hbm,
                     aggp_out, segp_out,
                     idxr_v, rowf_v, colf_v, mbuf_v, cbuf_v, tbuf_v, coord_v,
                     zbuf_v, agg_sh, seg_sh, sem):
    cid = lax.axis_index("c")
    sid = lax.axis_index("s")
    pltpu.sync_copy(coordf_hbm, coord_v)

    z16 = jnp.zeros((16,), _f32)

    def zrow(i, carry):
        for k in range(H // 16):
            zbuf_v[i, pl.ds(k * 16, 16)] = z16
        return carry
    lax.fori_loop(0, 32, zrow, 0)

    def trow(i, carry):
        tbuf_v[i, pl.ds(0, 16)] = z16
        return carry
    lax.fori_loop(0, CH, trow, 0)

    nper = NPAD // NS
    for k in range(nper // 32):
        pltpu.sync_copy(zbuf_v, agg_sh.at[pl.ds(sid * nper + k * 32, 32)])
        pltpu.sync_copy(zbuf_v.at[:, pl.ds(0, 16)],
                        seg_sh.at[pl.ds(sid * nper + k * 32, 32)])
    plsc.subcore_barrier()

    ones16 = jnp.full((16,), 1.0, _f32)
    lane16 = lax.iota(jnp.int32, 16)

    def chunk(ci, carry):
        off = (sid * NC + cid) * EPW + ci * CH
        for j in range(NSSC):
            pltpu.sync_copy(row_hbm.at[pl.ds(off + j * SSC, SSC)], idxr_v.at[j])
        pltpu.sync_copy(row_hbm.at[pl.ds(off, CH)], rowf_v)
        pltpu.sync_copy(col_hbm.at[pl.ds(off, CH)], colf_v)
        pltpu.sync_copy(c_hbm.at[pl.ds(off, CH)], cbuf_v)
        for g in range(CH // 16):
            r16 = rowf_v[pl.ds(g * 16, 16)]
            c16 = colf_v[pl.ds(g * 16, 16)]
            cval = cbuf_v[pl.ds(g * 16, 16)]
            eidx = lane16 + (g * 16)
            r3 = r16 * 3
            c3 = c16 * 3
            for d in range(3):
                dsp = jnp.full((16,), d, jnp.int32)
                pr = plsc.load_gather(coord_v, [r3 + dsp])
                pc = plsc.load_gather(coord_v, [c3 + dsp])
                plsc.store_scatter(tbuf_v, [eidx, dsp], (pr - pc) * cval)
            plsc.store_scatter(tbuf_v, [eidx, jnp.full((16,), 3, jnp.int32)],
                               ones16)
        for j in range(NSSC):
            pltpu.sync_copy(tbuf_v.at[pl.ds(j * SSC, SSC)],
                            seg_sh.at[idxr_v.at[j]], add=True)
        for half in range(2):
            pltpu.sync_copy(m_hbm.at[pl.ds(off + half * (CH // 2), CH // 2)],
                            mbuf_v)
            for j in range(NSSC // 2):
                pltpu.sync_copy(
                    mbuf_v.at[pl.ds(j * SSC, SSC)],
                    agg_sh.at[idxr_v.at[half * (NSSC // 2) + j]], add=True)
        return carry

    lax.fori_loop(0, NCHUNK, chunk, 0)
    plsc.subcore_barrier()

    for k in range(5):
        r0 = sid * nper + k * 128
        pltpu.sync_copy(agg_sh.at[pl.ds(r0, 128)], aggp_out.at[cid, pl.ds(r0, 128)])
        pltpu.sync_copy(seg_sh.at[pl.ds(r0, 128)], segp_out.at[cid, pl.ds(r0, 128)])


def _sc_scatter_call(m, c, row1, col1, coordf):
    f = pl.kernel(
        _sc_scatter_body,
        out_type=[
            jax.ShapeDtypeStruct((2, NPAD, H), _f32),
            jax.ShapeDtypeStruct((2, NPAD, 16), _f32),
        ],
        mesh=_mesh(),
        compiler_params=_SC_PARAMS,
        scratch_types=[
            pltpu.VMEM((NSSC, SSC), jnp.int32),
            pltpu.VMEM((CH,), jnp.int32),
            pltpu.VMEM((CH,), jnp.int32),
            pltpu.VMEM((CH // 2, H), _f32),
            pltpu.VMEM((CH,), _f32),
            pltpu.VMEM((CH, 16), _f32),
            pltpu.VMEM((NPAD * 3,), _f32),
            pltpu.VMEM((32, H), _f32),
            pltpu.VMEM_SHARED((NPAD, H), _f32),
            pltpu.VMEM_SHARED((NPAD, 16), _f32),
            pltpu.SemaphoreType.DMA,
        ],
    )
    return f(m, c, row1, col1, coordf)


def kernel(h, x, edges, emb_in_W, emb_in_b, edge_W1, edge_b1, edge_W2, edge_b2,
           node_W1, node_b1, node_W2, node_b2, coord_W1, coord_b1, coord_W2,
           emb_out_W, emb_out_b, head_W1, head_b1, head_W2, head_b2):
    row1 = edges[0]
    col1 = edges[1]
    coord3 = jnp.pad(x, ((0, NPAD - N), (0, 0)))
    hpad = jnp.pad(h, ((0, NPAD - N), (0, 0)))

    eye4 = jnp.eye(4, dtype=_f32)
    w1a = [edge_W1[i, :H, :] for i in range(L)]
    w1b = [edge_W1[i, H:2 * H, :] for i in range(L)]
    r4 = [jnp.kron(eye4, edge_W1[i, 2 * H, :][None, :]) for i in range(L)]
    b1t = [jnp.tile(edge_b1[i], 4)[None, :] for i in range(L)]
    w2bd = [jnp.kron(eye4, edge_W2[i]) for i in range(L)]
    b2t = [jnp.tile(edge_b2[i], 4)[None, :] for i in range(L)]
    cw1bd = [jnp.kron(eye4, coord_W1[i]) for i in range(L)]
    cb1t = [jnp.tile(coord_b1[i], 4)[None, :] for i in range(L)]
    cw2bd = [jnp.kron(eye4, coord_W2[i]) for i in range(L)]

    hh, a, b = _init_call(hpad, emb_in_W, emb_in_b[None, :], w1a[0], w1b[0])

    out = None
    for i in range(L):
        coordf = coord3.reshape(NPAD * 3)
        ar, bc, rad = _sc_gather_call(a, b, coordf, row1, col1)
        m4, c4 = _edge_call(ar.reshape(E4, 256), bc.reshape(E4, 256),
                            rad.reshape(E4, 4), r4[i], b1t[i], w2bd[i], b2t[i],
                            cw1bd[i], cb1t[i], cw2bd[i])
        m = m4.reshape(E, H)
        c = c4.reshape(E)
        aggp, segp = _sc_scatter_call(m, c, row1, col1, coordf)
        if i < L - 1:
            hh, coord3, a, b = _node_call(
                hh, coord3, aggp, segp, node_W1[i], node_b1[i][None, :],
                node_W2[i], node_b2[i][None, :], w1a[i + 1], w1b[i + 1])
        else:
            out = _last_call(
                hh, aggp, node_W1[i], node_b1[i][None, :], node_W2[i],
                node_b2[i][None, :], emb_out_W, emb_out_b[None, :],
                head_W1, head_b1[None, :], head_W2, head_b2[None, :])
    return out[:N]


# speedup vs baseline: 5.3167x; 1.3909x over previous
import functools

import jax
import jax.numpy as jnp
from jax import lax
from jax.experimental import pallas as pl
from jax.experimental.pallas import tpu as pltpu
from jax.experimental.pallas import tpu_sc as plsc

N = 10000
E = 320000
H = 64
L = 4
D_IN = 128
D_OUT = 20

NC = 2
NS = 16
NW = NC * NS
EPW = E // NW
CH = 400
NCHUNK = EPW // CH
SUB = 80
NSUB = CH // SUB
SSC = 40
NSSC = CH // SSC

E4 = E // 4
BE4 = 1600
NPAD = 10240
NB = 2048

_f32 = jnp.float32


def _silu(v):
    return v * jax.nn.sigmoid(v)


def _bdot(x, w):
    return jnp.dot(x.astype(jnp.bfloat16), w.astype(jnp.bfloat16),
                   preferred_element_type=_f32)


def _init_body(h_ref, w_ref, b_ref, wa_ref, wb_ref, hh_ref, a_ref, bb_ref):
    hh = _bdot(h_ref[...], w_ref[...]) + b_ref[...]
    hh_ref[...] = hh
    a_ref[...] = _bdot(hh, wa_ref[...])
    bb_ref[...] = _bdot(hh, wb_ref[...])


def _init_call(h, w, b, wa, wb):
    return pl.pallas_call(
        _init_body,
        grid=(NPAD // NB,),
        in_specs=[
            pl.BlockSpec((NB, D_IN), lambda i: (i, 0)),
            pl.BlockSpec((D_IN, H), lambda i: (0, 0)),
            pl.BlockSpec((1, H), lambda i: (0, 0)),
            pl.BlockSpec((H, H), lambda i: (0, 0)),
            pl.BlockSpec((H, H), lambda i: (0, 0)),
        ],
        out_specs=[
            pl.BlockSpec((NB, H), lambda i: (i, 0)),
            pl.BlockSpec((NB, H), lambda i: (i, 0)),
            pl.BlockSpec((NB, H), lambda i: (i, 0)),
        ],
        out_shape=[jax.ShapeDtypeStruct((NPAD, H), _f32)] * 3,
    )(h, w, b, wa, wb)


def _edge_body(ar_ref, bc_ref, rad_ref, r4_ref, b1_ref, w2_ref, b2_ref,
               cw1_ref, cb1_ref, cw2_ref, m_ref, c_ref):
    z = (ar_ref[...] + bc_ref[...]
         + _bdot(rad_ref[...], r4_ref[...])
         + b1_ref[...])
    m1 = _silu(z)
    m2 = _silu(_bdot(m1, w2_ref[...]) + b2_ref[...])
    c1 = _silu(_bdot(m2, cw1_ref[...]) + cb1_ref[...])
    m_ref[...] = m2
    c_ref[...] = _bdot(c1, cw2_ref[...])


def _edge_call(ar4, bc4, rad4, r4, b1t, w2bd, b2t, cw1bd, cb1t, cw2bd):
    return pl.pallas_call(
        _edge_body,
        grid=(E4 // BE4,),
        in_specs=[
            pl.BlockSpec((BE4, 256), lambda i: (i, 0)),
            pl.BlockSpec((BE4, 256), lambda i: (i, 0)),
            pl.BlockSpec((BE4, 4), lambda i: (i, 0)),
            pl.BlockSpec((4, 256), lambda i: (0, 0)),
            pl.BlockSpec((1, 256), lambda i: (0, 0)),
            pl.BlockSpec((256, 256), lambda i: (0, 0)),
            pl.BlockSpec((1, 256), lambda i: (0, 0)),
            pl.BlockSpec((256, 256), lambda i: (0, 0)),
            pl.BlockSpec((1, 256), lambda i: (0, 0)),
            pl.BlockSpec((256, 4), lambda i: (0, 0)),
        ],
        out_specs=[
            pl.BlockSpec((BE4, 256), lambda i: (i, 0)),
            pl.BlockSpec((BE4, 4), lambda i: (i, 0)),
        ],
        out_shape=[
            jax.ShapeDtypeStruct((E4, 256), _f32),
            jax.ShapeDtypeStruct((E4, 4), _f32),
        ],
    )(ar4, bc4, rad4, r4, b1t, w2bd, b2t, cw1bd, cb1t, cw2bd)


def _node_body(hh_ref, coord_ref, agg_ref, seg_ref, nw1_ref, nb1_ref, nw2_ref,
               nb2_ref, wa_ref, wb_ref, hho_ref, coordo_ref, a_ref, bb_ref):
    agg = agg_ref[0] + agg_ref[1]
    hh = hh_ref[...]
    zin = jnp.concatenate([hh, agg], axis=1)
    u = _silu(_bdot(zin, nw1_ref[...]) + nb1_ref[...])
    hhn = hh + _bdot(u, nw2_ref[...]) + nb2_ref[...]
    hho_ref[...] = hhn
    seg = seg_ref[0] + seg_ref[1]
    cnt = jnp.clip(seg[:, 3:4], 1.0, None)
    coordo_ref[...] = coord_ref[...] + seg[:, 0:3] / cnt
    a_ref[...] = _bdot(hhn, wa_ref[...])
    bb_ref[...] = _bdot(hhn, wb_ref[...])


def _node_call(hh, coord3, aggp, segp, nw1, nb1, nw2, nb2, wa, wb):
    return pl.pallas_call(
        _node_body,
        grid=(NPAD // NB,),
        in_specs=[
            pl.BlockSpec((NB, H), lambda i: (i, 0)),
            pl.BlockSpec((NB, 3), lambda i: (i, 0)),
            pl.BlockSpec((2, NB, H), lambda i: (0, i, 0)),
            pl.BlockSpec((2, NB, 16), lambda i: (0, i, 0)),
            pl.BlockSpec((2 * H, H), lambda i: (0, 0)),
            pl.BlockSpec((1, H), lambda i: (0, 0)),
            pl.BlockSpec((H, H), lambda i: (0, 0)),
            pl.BlockSpec((1, H), lambda i: (0, 0)),
            pl.BlockSpec((H, H), lambda i: (0, 0)),
            pl.BlockSpec((H, H), lambda i: (0, 0)),
        ],
        out_specs=[
            pl.BlockSpec((NB, H), lambda i: (i, 0)),
            pl.BlockSpec((NB, 3), lambda i: (i, 0)),
            pl.BlockSpec((NB, H), lambda i: (i, 0)),
            pl.BlockSpec((NB, H), lambda i: (i, 0)),
        ],
        out_shape=[
            jax.ShapeDtypeStruct((NPAD, H), _f32),
            jax.ShapeDtypeStruct((NPAD, 3), _f32),
            jax.ShapeDtypeStruct((NPAD, H), _f32),
            jax.ShapeDtypeStruct((NPAD, H), _f32),
        ],
    )(hh, coord3, aggp, segp, nw1, nb1, nw2, nb2, wa, wb)


def _last_body(hh_ref, agg_ref, nw1_ref, nb1_ref, nw2_ref, nb2_ref, eow_ref,
               eob_ref, hw1_ref, hb1_ref, hw2_ref, hb2_ref, out_ref):
    agg = agg_ref[0] + agg_ref[1]
    hh = hh_ref[...]
    zin = jnp.concatenate([hh, agg], axis=1)
    u = _silu(_bdot(zin, nw1_ref[...]) + nb1_ref[...])
    hhn = hh + _bdot(u, nw2_ref[...]) + nb2_ref[...]
    g = _bdot(hhn, eow_ref[...]) + eob_ref[...]
    r = jnp.maximum(_bdot(g, hw1_ref[...]) + hb1_ref[...], 0.0)
    out_ref[...] = _bdot(r, hw2_ref[...]) + hb2_ref[...]


def _last_call(hh, aggp, nw1, nb1, nw2, nb2, eow, eob, hw1, hb1, hw2, hb2):
    return pl.pallas_call(
        _last_body,
        grid=(NPAD // NB,),
        in_specs=[
            pl.BlockSpec((NB, H), lambda i: (i, 0)),
            pl.BlockSpec((2, NB, H), lambda i: (0, i, 0)),
            pl.BlockSpec((2 * H, H), lambda i: (0, 0)),
            pl.BlockSpec((1, H), lambda i: (0, 0)),
            pl.BlockSpec((H, H), lambda i: (0, 0)),
            pl.BlockSpec((1, H), lambda i: (0, 0)),
            pl.BlockSpec((H, H), lambda i: (0, 0)),
            pl.BlockSpec((1, H), lambda i: (0, 0)),
            pl.BlockSpec((H, H), lambda i: (0, 0)),
            pl.BlockSpec((1, H), lambda i: (0, 0)),
            pl.BlockSpec((H, D_OUT), lambda i: (0, 0)),
            pl.BlockSpec((1, D_OUT), lambda i: (0, 0)),
        ],
        out_specs=[pl.BlockSpec((NB, D_OUT), lambda i: (i, 0))],
        out_shape=[jax.ShapeDtypeStruct((NPAD, D_OUT), _f32)],
    )(hh, aggp, nw1, nb1, nw2, nb2, eow, eob, hw1, hb1, hw2, hb2)[0]


@functools.cache
def _mesh():
    return plsc.VectorSubcoreMesh(core_axis_name="c", subcore_axis_name="s",
                                  num_cores=NC, num_subcores=NS)


_SC_PARAMS = pltpu.CompilerParams(needs_layout_passes=False,
                                  use_tc_tiling_on_sc=False)


def _sc_gather_body(a_hbm, b_hbm, coordf_hbm, row_hbm, col_hbm,
                    ar_out, bc_out, rad_out,
                    rowall_v, colall_v, bufa_v, bufb_v, coord_v, rad_v, sem):
    wid = lax.axis_index("s") * NC + lax.axis_index("c")
    base = wid * EPW
    pltpu.sync_copy(coordf_hbm, coord_v)
    pltpu.sync_copy(row_hbm.at[pl.ds(base, EPW)], rowall_v)
    pltpu.sync_copy(col_hbm.at[pl.ds(base, EPW)], colall_v)

    def chunk(ci, carry):
        off = base + ci * CH
        loff = ci * CH
        cps = []
        for j in range(NSUB):
            cps.append(pltpu.async_copy(
                a_hbm.at[rowall_v.at[pl.ds(loff + j * SUB, SUB)]],
                bufa_v.at[pl.ds(j * SUB, SUB)], sem))
            cps.append(pltpu.async_copy(
                b_hbm.at[colall_v.at[pl.ds(loff + j * SUB, SUB)]],
                bufb_v.at[pl.ds(j * SUB, SUB)], sem))
        for g in range(CH // 16):
            r16 = rowall_v[pl.ds(loff + g * 16, 16)]
            c16 = colall_v[pl.ds(loff + g * 16, 16)]
            r3 = r16 * 3
            c3 = c16 * 3
            acc = jnp.zeros((16,), _f32)
            for d in range(3):
                dsp = jnp.full((16,), d, jnp.int32)
                pr = plsc.load_gather(coord_v, [r3 + dsp])
                pc = plsc.load_gather(coord_v, [c3 + dsp])
                df = pr - pc
                acc = acc + df * df
            rad_v[pl.ds(g * 16, 16)] = acc
        for cp in cps:
            cp.wait()
        pltpu.sync_copy(bufa_v, ar_out.at[pl.ds(off, CH)])
        pltpu.sync_copy(bufb_v, bc_out.at[pl.ds(off, CH)])
        pltpu.sync_copy(rad_v, rad_out.at[pl.ds(off, CH)])
        return carry

    lax.fori_loop(0, NCHUNK, chunk, 0)


def _sc_gather_call(a, b, coordf, row1, col1):
    f = pl.kernel(
        _sc_gather_body,
        out_type=[
            jax.ShapeDtypeStruct((E, H), _f32),
            jax.ShapeDtypeStruct((E, H), _f32),
            jax.ShapeDtypeStruct((E,), _f32),
        ],
        mesh=_mesh(),
        compiler_params=_SC_PARAMS,
        scratch_types=[
            pltpu.VMEM((EPW,), jnp.int32),
            pltpu.VMEM((EPW,), jnp.int32),
            pltpu.VMEM((CH, H), _f32),
            pltpu.VMEM((CH, H), _f32),
            pltpu.VMEM((NPAD * 3,), _f32),
            pltpu.VMEM((CH,), _f32),
            pltpu.SemaphoreType.DMA,
        ],
    )
    return f(a, b, coordf, row1, col1)


def _sc_scatter_body(m_hbm, c_hbm, row_hbm, col_hbm, coordf_hbm,
                     aggp_out, segp_out,
                     rowf_v, colf_v, mbuf_v, cbuf_v, tbuf_v, coord_v,
                     zbuf_v, agg_sh, seg_sh, sem, sem2):
    cid = lax.axis_index("c")
    sid = lax.axis_index("s")
    pltpu.sync_copy(coordf_hbm, coord_v)

    z16 = jnp.zeros((16,), _f32)

    def zrow(i, carry):
        for k in range(H // 16):
            zbuf_v[i, pl.ds(k * 16, 16)] = z16
        return carry
    lax.fori_loop(0, 32, zrow, 0)

    def trow(i, carry):
        tbuf_v[i, pl.ds(0, 16)] = z16
        return carry
    lax.fori_loop(0, CH, trow, 0)

    nper = NPAD // NS
    for k in range(nper // 32):
        pltpu.sync_copy(zbuf_v, agg_sh.at[pl.ds(sid * nper + k * 32, 32)])
        pltpu.sync_copy(zbuf_v.at[:, pl.ds(0, 16)],
                        seg_sh.at[pl.ds(sid * nper + k * 32, 32)])
    plsc.subcore_barrier()

    ones16 = jnp.full((16,), 1.0, _f32)
    lane16 = lax.iota(jnp.int32, 16)

    def chunk(ci, carry):
        off = (sid * NC + cid) * EPW + ci * CH
        mcp = pltpu.async_copy(m_hbm.at[pl.ds(off, CH)], mbuf_v, sem2)
        pltpu.sync_copy(row_hbm.at[pl.ds(off, CH)], rowf_v)
        pltpu.sync_copy(col_hbm.at[pl.ds(off, CH)], colf_v)
        pltpu.sync_copy(c_hbm.at[pl.ds(off, CH)], cbuf_v)
        for g in range(CH // 16):
            r16 = rowf_v[pl.ds(g * 16, 16)]
            c16 = colf_v[pl.ds(g * 16, 16)]
            cval = cbuf_v[pl.ds(g * 16, 16)]
            eidx = lane16 + (g * 16)
            r3 = r16 * 3
            c3 = c16 * 3
            for d in range(3):
                dsp = jnp.full((16,), d, jnp.int32)
                pr = plsc.load_gather(coord_v, [r3 + dsp])
                pc = plsc.load_gather(coord_v, [c3 + dsp])
                plsc.store_scatter(tbuf_v, [eidx, dsp], (pr - pc) * cval)
            plsc.store_scatter(tbuf_v, [eidx, jnp.full((16,), 3, jnp.int32)],
                               ones16)
        mcp.wait()
        cps = []
        for j in range(NSSC):
            cps.append(pltpu.async_copy(
                tbuf_v.at[pl.ds(j * SSC, SSC)],
                seg_sh.at[rowf_v.at[pl.ds(j * SSC, SSC)]], sem, add=True))
            cps.append(pltpu.async_copy(
                mbuf_v.at[pl.ds(j * SSC, SSC)],
                agg_sh.at[rowf_v.at[pl.ds(j * SSC, SSC)]], sem, add=True))
        for cp in cps:
            cp.wait()
        return carry

    lax.fori_loop(0, NCHUNK, chunk, 0)
    plsc.subcore_barrier()

    for k in range(5):
        r0 = sid * nper + k * 128
        pltpu.sync_copy(agg_sh.at[pl.ds(r0, 128)], aggp_out.at[cid, pl.ds(r0, 128)])
        pltpu.sync_copy(seg_sh.at[pl.ds(r0, 128)], segp_out.at[cid, pl.ds(r0, 128)])


def _sc_scatter_call(m, c, row1, col1, coordf):
    f = pl.kernel(
        _sc_scatter_body,
        out_type=[
            jax.ShapeDtypeStruct((2, NPAD, H), _f32),
            jax.ShapeDtypeStruct((2, NPAD, 16), _f32),
        ],
        mesh=_mesh(),
        compiler_params=_SC_PARAMS,
        scratch_types=[
            pltpu.VMEM((CH,), jnp.int32),
            pltpu.VMEM((CH,), jnp.int32),
            pltpu.VMEM((CH, H), _f32),
            pltpu.VMEM((CH,), _f32),
            pltpu.VMEM((CH, 16), _f32),
            pltpu.VMEM((NPAD * 3,), _f32),
            pltpu.VMEM((32, H), _f32),
            pltpu.VMEM_SHARED((NPAD, H), _f32),
            pltpu.VMEM_SHARED((NPAD, 16), _f32),
            pltpu.SemaphoreType.DMA,
            pltpu.SemaphoreType.DMA,
        ],
    )
    return f(m, c, row1, col1, coordf)


def kernel(h, x, edges, emb_in_W, emb_in_b, edge_W1, edge_b1, edge_W2, edge_b2,
           node_W1, node_b1, node_W2, node_b2, coord_W1, coord_b1, coord_W2,
           emb_out_W, emb_out_b, head_W1, head_b1, head_W2, head_b2):
    row1 = edges[0]
    col1 = edges[1]
    coord3 = jnp.pad(x, ((0, NPAD - N), (0, 0)))
    hpad = jnp.pad(h, ((0, NPAD - N), (0, 0)))

    eye4 = jnp.eye(4, dtype=_f32)
    w1a = [edge_W1[i, :H, :] for i in range(L)]
    w1b = [edge_W1[i, H:2 * H, :] for i in range(L)]
    r4 = [jnp.kron(eye4, edge_W1[i, 2 * H, :][None, :]) for i in range(L)]
    b1t = [jnp.tile(edge_b1[i], 4)[None, :] for i in range(L)]
    w2bd = [jnp.kron(eye4, edge_W2[i]) for i in range(L)]
    b2t = [jnp.tile(edge_b2[i], 4)[None, :] for i in range(L)]
    cw1bd = [jnp.kron(eye4, coord_W1[i]) for i in range(L)]
    cb1t = [jnp.tile(coord_b1[i], 4)[None, :] for i in range(L)]
    cw2bd = [jnp.kron(eye4, coord_W2[i]) for i in range(L)]

    hh, a, b = _init_call(hpad, emb_in_W, emb_in_b[None, :], w1a[0], w1b[0])

    out = None
    for i in range(L):
        coordf = coord3.reshape(NPAD * 3)
        ar, bc, rad = _sc_gather_call(a, b, coordf, row1, col1)
        m4, c4 = _edge_call(ar.reshape(E4, 256), bc.reshape(E4, 256),
                            rad.reshape(E4, 4), r4[i], b1t[i], w2bd[i], b2t[i],
                            cw1bd[i], cb1t[i], cw2bd[i])
        m = m4.reshape(E, H)
        c = c4.reshape(E)
        aggp, segp = _sc_scatter_call(m, c, row1, col1, coordf)
        if i < L - 1:
            hh, coord3, a, b = _node_call(
                hh, coord3, aggp, segp, node_W1[i], node_b1[i][None, :],
                node_W2[i], node_b2[i][None, :], w1a[i + 1], w1b[i + 1])
        else:
            out = _last_call(
                hh, aggp, node_W1[i], node_b1[i][None, :], node_W2[i],
                node_b2[i][None, :], emb_out_W, emb_out_b[None, :],
                head_W1, head_b1[None, :], head_W2, head_b2[None, :])
    return out[:N]


# speedup vs baseline: 5.3381x; 1.0040x over previous
import functools

import jax
import jax.numpy as jnp
from jax import lax
from jax.experimental import pallas as pl
from jax.experimental.pallas import tpu as pltpu
from jax.experimental.pallas import tpu_sc as plsc

N = 10000
E = 320000
H = 64
L = 4
D_IN = 128
D_OUT = 20

NC = 2
NS = 16
NW = NC * NS
EPW = E // NW
CH = 400
NCHUNK = EPW // CH
SUB = 80
NSUB = CH // SUB
SSC = 40
NSSC = CH // SSC

E4 = E // 4
BE4 = 1600
NPAD = 10240
NB = 2048

_f32 = jnp.float32


def _silu(v):
    return v * jax.nn.sigmoid(v)


def _bdot(x, w):
    return jnp.dot(x.astype(jnp.bfloat16), w.astype(jnp.bfloat16),
                   preferred_element_type=_f32)


def _init_body(h_ref, w_ref, b_ref, wa_ref, wb_ref, hh_ref, a_ref, bb_ref):
    hh = _bdot(h_ref[...], w_ref[...]) + b_ref[...]
    hh_ref[...] = hh
    a_ref[...] = _bdot(hh, wa_ref[...])
    bb_ref[...] = _bdot(hh, wb_ref[...])


def _init_call(h, w, b, wa, wb):
    return pl.pallas_call(
        _init_body,
        grid=(NPAD // NB,),
        in_specs=[
            pl.BlockSpec((NB, D_IN), lambda i: (i, 0)),
            pl.BlockSpec((D_IN, H), lambda i: (0, 0)),
            pl.BlockSpec((1, H), lambda i: (0, 0)),
            pl.BlockSpec((H, H), lambda i: (0, 0)),
            pl.BlockSpec((H, H), lambda i: (0, 0)),
        ],
        out_specs=[
            pl.BlockSpec((NB, H), lambda i: (i, 0)),
            pl.BlockSpec((NB, H), lambda i: (i, 0)),
            pl.BlockSpec((NB, H), lambda i: (i, 0)),
        ],
        out_shape=[jax.ShapeDtypeStruct((NPAD, H), _f32)] * 3,
    )(h, w, b, wa, wb)


def _edge_body(ar_ref, bc_ref, rad_ref, r4_ref, b1_ref, w2_ref, b2_ref,
               cw1_ref, cb1_ref, cw2_ref, m_ref, c_ref):
    z = (ar_ref[...] + bc_ref[...]
         + _bdot(rad_ref[...], r4_ref[...])
         + b1_ref[...])
    m1 = _silu(z)
    m2 = _silu(_bdot(m1, w2_ref[...]) + b2_ref[...])
    c1 = _silu(_bdot(m2, cw1_ref[...]) + cb1_ref[...])
    m_ref[...] = m2
    c_ref[...] = _bdot(c1, cw2_ref[...])


def _edge_call(ar4, bc4, rad4, r4, b1t, w2bd, b2t, cw1bd, cb1t, cw2bd):
    return pl.pallas_call(
        _edge_body,
        grid=(E4 // BE4,),
        in_specs=[
            pl.BlockSpec((BE4, 256), lambda i: (i, 0)),
            pl.BlockSpec((BE4, 256), lambda i: (i, 0)),
            pl.BlockSpec((BE4, 4), lambda i: (i, 0)),
            pl.BlockSpec((4, 256), lambda i: (0, 0)),
            pl.BlockSpec((1, 256), lambda i: (0, 0)),
            pl.BlockSpec((256, 256), lambda i: (0, 0)),
            pl.BlockSpec((1, 256), lambda i: (0, 0)),
            pl.BlockSpec((256, 256), lambda i: (0, 0)),
            pl.BlockSpec((1, 256), lambda i: (0, 0)),
            pl.BlockSpec((256, 4), lambda i: (0, 0)),
        ],
        out_specs=[
            pl.BlockSpec((BE4, 256), lambda i: (i, 0)),
            pl.BlockSpec((BE4, 4), lambda i: (i, 0)),
        ],
        out_shape=[
            jax.ShapeDtypeStruct((E4, 256), _f32),
            jax.ShapeDtypeStruct((E4, 4), _f32),
        ],
    )(ar4, bc4, rad4, r4, b1t, w2bd, b2t, cw1bd, cb1t, cw2bd)


def _node_body(hh_ref, coord_ref, agg_ref, seg_ref, nw1_ref, nb1_ref, nw2_ref,
               nb2_ref, wa_ref, wb_ref, hho_ref, coordo_ref, a_ref, bb_ref):
    agg = agg_ref[0] + agg_ref[1]
    hh = hh_ref[...]
    zin = jnp.concatenate([hh, agg], axis=1)
    u = _silu(_bdot(zin, nw1_ref[...]) + nb1_ref[...])
    hhn = hh + _bdot(u, nw2_ref[...]) + nb2_ref[...]
    hho_ref[...] = hhn
    seg = seg_ref[0] + seg_ref[1]
    cnt = jnp.clip(seg[:, 3:4], 1.0, None)
    coordo_ref[...] = coord_ref[...] + seg[:, 0:3] / cnt
    a_ref[...] = _bdot(hhn, wa_ref[...])
    bb_ref[...] = _bdot(hhn, wb_ref[...])


def _node_call(hh, coord3, aggp, segp, nw1, nb1, nw2, nb2, wa, wb):
    return pl.pallas_call(
        _node_body,
        grid=(NPAD // NB,),
        in_specs=[
            pl.BlockSpec((NB, H), lambda i: (i, 0)),
            pl.BlockSpec((NB, 3), lambda i: (i, 0)),
            pl.BlockSpec((2, NB, H), lambda i: (0, i, 0)),
            pl.BlockSpec((2, NB, 16), lambda i: (0, i, 0)),
            pl.BlockSpec((2 * H, H), lambda i: (0, 0)),
            pl.BlockSpec((1, H), lambda i: (0, 0)),
            pl.BlockSpec((H, H), lambda i: (0, 0)),
            pl.BlockSpec((1, H), lambda i: (0, 0)),
            pl.BlockSpec((H, H), lambda i: (0, 0)),
            pl.BlockSpec((H, H), lambda i: (0, 0)),
        ],
        out_specs=[
            pl.BlockSpec((NB, H), lambda i: (i, 0)),
            pl.BlockSpec((NB, 3), lambda i: (i, 0)),
            pl.BlockSpec((NB, H), lambda i: (i, 0)),
            pl.BlockSpec((NB, H), lambda i: (i, 0)),
        ],
        out_shape=[
            jax.ShapeDtypeStruct((NPAD, H), _f32),
            jax.ShapeDtypeStruct((NPAD, 3), _f32),
            jax.ShapeDtypeStruct((NPAD, H), _f32),
            jax.ShapeDtypeStruct((NPAD, H), _f32),
        ],
    )(hh, coord3, aggp, segp, nw1, nb1, nw2, nb2, wa, wb)


def _last_body(hh_ref, agg_ref, nw1_ref, nb1_ref, nw2_ref, nb2_ref, eow_ref,
               eob_ref, hw1_ref, hb1_ref, hw2_ref, hb2_ref, out_ref):
    agg = agg_ref[0] + agg_ref[1]
    hh = hh_ref[...]
    zin = jnp.concatenate([hh, agg], axis=1)
    u = _silu(_bdot(zin, nw1_ref[...]) + nb1_ref[...])
    hhn = hh + _bdot(u, nw2_ref[...]) + nb2_ref[...]
    g = _bdot(hhn, eow_ref[...]) + eob_ref[...]
    r = jnp.maximum(_bdot(g, hw1_ref[...]) + hb1_ref[...], 0.0)
    out_ref[...] = _bdot(r, hw2_ref[...]) + hb2_ref[...]


def _last_call(hh, aggp, nw1, nb1, nw2, nb2, eow, eob, hw1, hb1, hw2, hb2):
    return pl.pallas_call(
        _last_body,
        grid=(NPAD // NB,),
        in_specs=[
            pl.BlockSpec((NB, H), lambda i: (i, 0)),
            pl.BlockSpec((2, NB, H), lambda i: (0, i, 0)),
            pl.BlockSpec((2 * H, H), lambda i: (0, 0)),
            pl.BlockSpec((1, H), lambda i: (0, 0)),
            pl.BlockSpec((H, H), lambda i: (0, 0)),
            pl.BlockSpec((1, H), lambda i: (0, 0)),
            pl.BlockSpec((H, H), lambda i: (0, 0)),
            pl.BlockSpec((1, H), lambda i: (0, 0)),
            pl.BlockSpec((H, H), lambda i: (0, 0)),
            pl.BlockSpec((1, H), lambda i: (0, 0)),
            pl.BlockSpec((H, D_OUT), lambda i: (0, 0)),
            pl.BlockSpec((1, D_OUT), lambda i: (0, 0)),
        ],
        out_specs=[pl.BlockSpec((NB, D_OUT), lambda i: (i, 0))],
        out_shape=[jax.ShapeDtypeStruct((NPAD, D_OUT), _f32)],
    )(hh, aggp, nw1, nb1, nw2, nb2, eow, eob, hw1, hb1, hw2, hb2)[0]


@functools.cache
def _mesh():
    return plsc.VectorSubcoreMesh(core_axis_name="c", subcore_axis_name="s",
                                  num_cores=NC, num_subcores=NS)


_SC_PARAMS = pltpu.CompilerParams(needs_layout_passes=False,
                                  use_tc_tiling_on_sc=False)


def _sc_gather_body(a_hbm, b_hbm, coordf_hbm, row_hbm, col_hbm,
                    ar_out, bc_out, rad_out,
                    rowall_v, colall_v, bufa_v, bufb_v, coord_v, rad_v, sem):
    wid = lax.axis_index("s") * NC + lax.axis_index("c")
    base = wid * EPW
    pltpu.sync_copy(coordf_hbm, coord_v)
    pltpu.sync_copy(row_hbm.at[pl.ds(base, EPW)], rowall_v)
    pltpu.sync_copy(col_hbm.at[pl.ds(base, EPW)], colall_v)

    def chunk(ci, carry):
        off = base + ci * CH
        loff = ci * CH
        cps = []
        for j in range(NSUB):
            cps.append(pltpu.async_copy(
                a_hbm.at[rowall_v.at[pl.ds(loff + j * SUB, SUB)]],
                bufa_v.at[pl.ds(j * SUB, SUB)], sem))
            cps.append(pltpu.async_copy(
                b_hbm.at[colall_v.at[pl.ds(loff + j * SUB, SUB)]],
                bufb_v.at[pl.ds(j * SUB, SUB)], sem))
        for g in range(CH // 16):
            r16 = rowall_v[pl.ds(loff + g * 16, 16)]
            c16 = colall_v[pl.ds(loff + g * 16, 16)]
            r3 = r16 * 3
            c3 = c16 * 3
            acc = jnp.zeros((16,), _f32)
            for d in range(3):
                dsp = jnp.full((16,), d, jnp.int32)
                pr = plsc.load_gather(coord_v, [r3 + dsp])
                pc = plsc.load_gather(coord_v, [c3 + dsp])
                df = pr - pc
                acc = acc + df * df
            rad_v[pl.ds(g * 16, 16)] = acc
        for cp in cps:
            cp.wait()
        pltpu.sync_copy(bufa_v, ar_out.at[pl.ds(off, CH)])
        pltpu.sync_copy(bufb_v, bc_out.at[pl.ds(off, CH)])
        pltpu.sync_copy(rad_v, rad_out.at[pl.ds(off, CH)])
        return carry

    lax.fori_loop(0, NCHUNK, chunk, 0)


def _sc_gather_call(a, b, coordf, row1, col1):
    f = pl.kernel(
        _sc_gather_body,
        out_type=[
            jax.ShapeDtypeStruct((E, H), _f32),
            jax.ShapeDtypeStruct((E, H), _f32),
            jax.ShapeDtypeStruct((E,), _f32),
        ],
        mesh=_mesh(),
        compiler_params=_SC_PARAMS,
        scratch_types=[
            pltpu.VMEM((EPW,), jnp.int32),
            pltpu.VMEM((EPW,), jnp.int32),
            pltpu.VMEM((CH, H), _f32),
            pltpu.VMEM((CH, H), _f32),
            pltpu.VMEM((NPAD * 3,), _f32),
            pltpu.VMEM((CH,), _f32),
            pltpu.SemaphoreType.DMA,
        ],
    )
    return f(a, b, coordf, row1, col1)


def _sc_scatter_body(m_hbm, c_hbm, row_hbm, col_hbm, coordf_hbm,
                     aggp_out, segp_out,
                     rowf_v, colf_v, mbuf_v, cbuf_v, tbuf_v, coord_v,
                     zbuf_v, agg_sh, seg_sh, sem, sem2):
    cid = lax.axis_index("c")
    sid = lax.axis_index("s")
    pltpu.sync_copy(coordf_hbm, coord_v)

    z16 = jnp.zeros((16,), _f32)

    def zrow(i, carry):
        for k in range(H // 16):
            zbuf_v[i, pl.ds(k * 16, 16)] = z16
        return carry
    lax.fori_loop(0, 32, zrow, 0)

    def trow(i, carry):
        tbuf_v[i, pl.ds(0, 16)] = z16
        return carry
    lax.fori_loop(0, CH, trow, 0)

    nper = NPAD // NS
    for k in range(nper // 32):
        pltpu.sync_copy(zbuf_v, agg_sh.at[pl.ds(sid * nper + k * 32, 32)])
        pltpu.sync_copy(zbuf_v.at[:, pl.ds(0, 16)],
                        seg_sh.at[pl.ds(sid * nper + k * 32, 32)])
    plsc.subcore_barrier()

    ones16 = jnp.full((16,), 1.0, _f32)
    lane16 = lax.iota(jnp.int32, 16)

    def batch(bi, carry):
        boff = (sid * NC + cid) * EPW + bi * (5 * CH)
        pltpu.sync_copy(row_hbm.at[pl.ds(boff, 5 * CH)], rowf_v)
        pltpu.sync_copy(col_hbm.at[pl.ds(boff, 5 * CH)], colf_v)
        pltpu.sync_copy(c_hbm.at[pl.ds(boff, 5 * CH)], cbuf_v)
        for ck in range(5):
            off = boff + ck * CH
            lo = ck * CH
            mcp = pltpu.async_copy(m_hbm.at[pl.ds(off, CH)], mbuf_v, sem2)
            for g in range(CH // 16):
                r16 = rowf_v[pl.ds(lo + g * 16, 16)]
                c16 = colf_v[pl.ds(lo + g * 16, 16)]
                cval = cbuf_v[pl.ds(lo + g * 16, 16)]
                eidx = lane16 + (g * 16)
                r3 = r16 * 3
                c3 = c16 * 3
                for d in range(3):
                    dsp = jnp.full((16,), d, jnp.int32)
                    pr = plsc.load_gather(coord_v, [r3 + dsp])
                    pc = plsc.load_gather(coord_v, [c3 + dsp])
                    plsc.store_scatter(tbuf_v, [eidx, dsp], (pr - pc) * cval)
                plsc.store_scatter(tbuf_v, [eidx, jnp.full((16,), 3, jnp.int32)],
                                   ones16)
            mcp.wait()
            cps = []
            for j in range(NSSC):
                cps.append(pltpu.async_copy(
                    tbuf_v.at[pl.ds(j * SSC, SSC)],
                    seg_sh.at[rowf_v.at[pl.ds(lo + j * SSC, SSC)]], sem, add=True))
                cps.append(pltpu.async_copy(
                    mbuf_v.at[pl.ds(j * SSC, SSC)],
                    agg_sh.at[rowf_v.at[pl.ds(lo + j * SSC, SSC)]], sem, add=True))
            for cp in cps:
                cp.wait()
        return carry

    lax.fori_loop(0, NCHUNK // 5, batch, 0)
    plsc.subcore_barrier()

    for k in range(5):
        r0 = sid * nper + k * 128
        pltpu.sync_copy(agg_sh.at[pl.ds(r0, 128)], aggp_out.at[cid, pl.ds(r0, 128)])
        pltpu.sync_copy(seg_sh.at[pl.ds(r0, 128)], segp_out.at[cid, pl.ds(r0, 128)])


def _sc_scatter_call(m, c, row1, col1, coordf):
    f = pl.kernel(
        _sc_scatter_body,
        out_type=[
            jax.ShapeDtypeStruct((2, NPAD, H), _f32),
            jax.ShapeDtypeStruct((2, NPAD, 16), _f32),
        ],
        mesh=_mesh(),
        compiler_params=_SC_PARAMS,
        scratch_types=[
            pltpu.VMEM((5 * CH,), jnp.int32),
            pltpu.VMEM((5 * CH,), jnp.int32),
            pltpu.VMEM((CH, H), _f32),
            pltpu.VMEM((5 * CH,), _f32),
            pltpu.VMEM((CH, 16), _f32),
            pltpu.VMEM((NPAD * 3,), _f32),
            pltpu.VMEM((32, H), _f32),
            pltpu.VMEM_SHARED((NPAD, H), _f32),
            pltpu.VMEM_SHARED((NPAD, 16), _f32),
            pltpu.SemaphoreType.DMA,
            pltpu.SemaphoreType.DMA,
        ],
    )
    return f(m, c, row1, col1, coordf)


def kernel(h, x, edges, emb_in_W, emb_in_b, edge_W1, edge_b1, edge_W2, edge_b2,
           node_W1, node_b1, node_W2, node_b2, coord_W1, coord_b1, coord_W2,
           emb_out_W, emb_out_b, head_W1, head_b1, head_W2, head_b2):
    row1 = edges[0]
    col1 = edges[1]
    coord3 = jnp.pad(x, ((0, NPAD - N), (0, 0)))
    hpad = jnp.pad(h, ((0, NPAD - N), (0, 0)))

    eye4 = jnp.eye(4, dtype=_f32)
    w1a = [edge_W1[i, :H, :] for i in range(L)]
    w1b = [edge_W1[i, H:2 * H, :] for i in range(L)]
    r4 = [jnp.kron(eye4, edge_W1[i, 2 * H, :][None, :]) for i in range(L)]
    b1t = [jnp.tile(edge_b1[i], 4)[None, :] for i in range(L)]
    w2bd = [jnp.kron(eye4, edge_W2[i]) for i in range(L)]
    b2t = [jnp.tile(edge_b2[i], 4)[None, :] for i in range(L)]
    cw1bd = [jnp.kron(eye4, coord_W1[i]) for i in range(L)]
    cb1t = [jnp.tile(coord_b1[i], 4)[None, :] for i in range(L)]
    cw2bd = [jnp.kron(eye4, coord_W2[i]) for i in range(L)]

    hh, a, b = _init_call(hpad, emb_in_W, emb_in_b[None, :], w1a[0], w1b[0])

    out = None
    for i in range(L):
        coordf = coord3.reshape(NPAD * 3)
        ar, bc, rad = _sc_gather_call(a, b, coordf, row1, col1)
        m4, c4 = _edge_call(ar.reshape(E4, 256), bc.reshape(E4, 256),
                            rad.reshape(E4, 4), r4[i], b1t[i], w2bd[i], b2t[i],
                            cw1bd[i], cb1t[i], cw2bd[i])
        m = m4.reshape(E, H)
        c = c4.reshape(E)
        aggp, segp = _sc_scatter_call(m, c, row1, col1, coordf)
        if i < L - 1:
            hh, coord3, a, b = _node_call(
                hh, coord3, aggp, segp, node_W1[i], node_b1[i][None, :],
                node_W2[i], node_b2[i][None, :], w1a[i + 1], w1b[i + 1])
        else:
            out = _last_call(
                hh, aggp, node_W1[i], node_b1[i][None, :], node_W2[i],
                node_b2[i][None, :], emb_out_W, emb_out_b[None, :],
                head_W1, head_b1[None, :], head_W2, head_b2[None, :])
    return out[:N]
